# baseline (speedup 1.0000x reference)
import jax
import jax.numpy as jnp
from jax import lax
from jax.experimental import pallas as pl
from jax.experimental.pallas import tpu as pltpu
from jax.experimental.pallas import tpu_sc as plsc

CONF_THRES = 0.25
IOU_THRES = 0.45
MAX_DET = 300
N_BOXES = 20000
PAD_N = 20480
NSUB = 16
CHUNK = PAD_N // NSUB
L = 16
STEPS = CHUNK // L


def _nms_sc(cx_hbm, cy_hbm, w_hbm, h_hbm, s_hbm, out_hbm,
            x1v, y1v, x2v, y2v, actv, pubv, mrgv, outv, shared):
    cid = lax.axis_index("c")
    sid = lax.axis_index("s")
    base = sid * CHUNK
    first = jnp.logical_and(cid == 0, sid == 0)

    pltpu.sync_copy(cx_hbm, x1v)
    pltpu.sync_copy(cy_hbm, y1v)
    pltpu.sync_copy(w_hbm, x2v)
    pltpu.sync_copy(h_hbm, y2v)
    pltpu.sync_copy(s_hbm.at[pl.ds(base, CHUNK)], actv)

    iota = lax.broadcasted_iota(jnp.int32, (L,), 0)

    def _perm(x, idx):
        return x.at[idx].get(mode="promise_in_bounds")

    def _xmax(x):
        for sh in (8, 4, 2, 1):
            x = jnp.maximum(x, _perm(x, iota ^ sh))
        return x

    def _xmin(x):
        for sh in (8, 4, 2, 1):
            x = jnp.minimum(x, _perm(x, iota ^ sh))
        return x

    def init_xyxy(k, c):
        sl = pl.ds(k * L, L)
        cx = x1v[sl] * 640.0
        cy = y1v[sl] * 640.0
        w = x2v[sl] * 100.0 + 2.0
        h = y2v[sl] * 100.0 + 2.0
        x1v[sl] = cx - w * 0.5
        y1v[sl] = cy - h * 0.5
        x2v[sl] = cx + w * 0.5
        y2v[sl] = cy + h * 0.5
        return c
    lax.fori_loop(0, PAD_N // L, init_xyxy, 0)

    @plsc.parallel_loop(0, STEPS, unroll=4)
    def _init_chunk(k):
        sl = pl.ds(k * L, L)
        s = actv[sl]
        actv[sl] = jnp.where(s > CONF_THRES, s, -1.0)

    bv0 = jnp.full((L,), -3e38, jnp.float32)
    bk0 = jnp.zeros((L,), jnp.int32)

    def fused_sweep(v, j, bx1, by1, bx2, by2, a1):
        @plsc.parallel_loop(0, STEPS, step=4,
                            carry=((bv0, bk0),) * 4, unroll=2)
        def chains(k0, am):
            out = []
            for c in range(4):
                bv2, bk2 = am[c]
                k = k0 + c
                sl = pl.ds(k * L, L)
                gsl = pl.ds(base + k * L, L)
                x1 = x1v[gsl]
                y1 = y1v[gsl]
                x2 = x2v[gsl]
                y2 = y2v[gsl]
                xx1 = jnp.maximum(bx1, x1)
                yy1 = jnp.maximum(by1, y1)
                xx2 = jnp.minimum(bx2, x2)
                yy2 = jnp.minimum(by2, y2)
                inter = (jnp.maximum(xx2 - xx1, 0.0)
                         * jnp.maximum(yy2 - yy1, 0.0))
                a2 = (x2 - x1) * (y2 - y1)
                iou = inter / (a1 + a2 - inter + 1e-7)
                g = base + k * L + iota
                sup = jnp.logical_and(
                    jnp.logical_or(iou > IOU_THRES, g == j), v)
                nact = jnp.where(sup, -1.0, actv[sl])
                actv[sl] = nact
                upd = nact > bv2
                out.append((jnp.where(upd, nact, bv2),
                            jnp.where(upd, k, bk2)))
            return tuple(out)

        def comb(p, q):
            bvp, bkp = p
            bvq, bkq = q
            upd = (bvq > bvp) | ((bvq == bvp) & (bkq < bkp))
            return (jnp.where(upd, bvq, bvp), jnp.where(upd, bkq, bkp))
        (p0, p1, p2, p3) = chains
        return comb(comb(p0, p1), comb(p2, p3))

    vfalse = iota < 0
    j0 = jnp.zeros((L,), jnp.int32)
    c0 = plsc.load_gather(x1v, [j0])
    am_init = fused_sweep(vfalse, j0, c0, c0, c0, c0, c0)

    def round_body(i, am):
        bv, bk = am
        gidx_lane = base + bk * L + iota
        m_loc = _xmax(bv)
        j_loc = _xmin(jnp.where(bv == m_loc, gidx_lane, jnp.int32(2**30)))

        pubv[:] = jnp.where(iota == 0, m_loc,
                  jnp.where(iota == 1, j_loc.astype(jnp.float32), 0.0))
        par = (i & 1) * (NSUB * L)
        pltpu.sync_copy(pubv, shared.at[pl.ds(par + sid * L, L)])
        plsc.subcore_barrier()
        pltpu.sync_copy(shared.at[pl.ds(par, NSUB * L)], mrgv)
        vals = plsc.load_gather(mrgv, [iota * L])
        idxs = plsc.load_gather(mrgv, [iota * L + 1])
        best_m = _xmax(vals)
        j = _xmin(jnp.where(vals == best_m, idxs, 3e38)).astype(jnp.int32)
        v = best_m > 0.0

        bx1 = plsc.load_gather(x1v, [j])
        by1 = plsc.load_gather(y1v, [j])
        bx2 = plsc.load_gather(x2v, [j])
        by2 = plsc.load_gather(y2v, [j])
        a1 = (bx2 - bx1) * (by2 - by1)

        @pl.when(first)
        def _():
            vf = jnp.where(v, 1.0, 0.0)
            row = jnp.where(iota == 0, bx1 * vf,
                  jnp.where(iota == 1, by1 * vf,
                  jnp.where(iota == 2, bx2 * vf,
                  jnp.where(iota == 3, by2 * vf,
                  jnp.where(iota == 4, best_m * vf, 0.0)))))
            plsc.store_scatter(outv, [i * L + iota], row)

        am_next = fused_sweep(v, j, bx1, by1, bx2, by2, a1)
        return am_next

    lax.fori_loop(0, MAX_DET, round_body, am_init)

    @pl.when(first)
    def _():
        pltpu.sync_copy(outv, out_hbm)


@jax.jit
def kernel(boxes, scores):
    pad = PAD_N - N_BOXES
    bp = jnp.pad(boxes, ((0, pad), (0, 0)))
    sp = jnp.pad(scores, (0, pad))
    f32 = jnp.float32
    mesh = plsc.VectorSubcoreMesh(core_axis_name="c", subcore_axis_name="s")
    k = pl.kernel(
        _nms_sc,
        mesh=mesh,
        compiler_params=pltpu.CompilerParams(needs_layout_passes=False),
        out_type=jax.ShapeDtypeStruct((MAX_DET * L,), f32),
        scratch_types=[
            pltpu.VMEM((PAD_N,), f32),
            pltpu.VMEM((PAD_N,), f32),
            pltpu.VMEM((PAD_N,), f32),
            pltpu.VMEM((PAD_N,), f32),
            pltpu.VMEM((CHUNK,), f32),
            pltpu.VMEM((L,), f32),
            pltpu.VMEM((NSUB * L,), f32),
            pltpu.VMEM((MAX_DET * L,), f32),
            pltpu.VMEM_SHARED((2 * NSUB * L,), f32),
        ],
    )
    out = k(bp[:, 0], bp[:, 1], bp[:, 2], bp[:, 3], sp)
    return out.reshape(MAX_DET, L)[:, :5]


# speedup vs baseline: 1.0191x; 1.0191x over previous
import jax
import jax.numpy as jnp
from jax import lax
from jax.experimental import pallas as pl
from jax.experimental.pallas import tpu as pltpu
from jax.experimental.pallas import tpu_sc as plsc

CONF_THRES = 0.25
IOU_THRES = 0.45
MAX_DET = 300
N_BOXES = 20000
PAD_N = 20480
NSUB = 16
CHUNK = PAD_N // NSUB
L = 16
STEPS = CHUNK // L


def _nms_sc(cx_hbm, cy_hbm, w_hbm, h_hbm, s_hbm, out_hbm,
            x1v, y1v, x2v, y2v, actv, pubv, mrgv, outv, shared):
    cid = lax.axis_index("c")
    sid = lax.axis_index("s")
    base = sid * CHUNK
    first = jnp.logical_and(cid == 0, sid == 0)

    pltpu.sync_copy(cx_hbm, x1v)
    pltpu.sync_copy(cy_hbm, y1v)
    pltpu.sync_copy(w_hbm, x2v)
    pltpu.sync_copy(h_hbm, y2v)
    pltpu.sync_copy(s_hbm.at[pl.ds(base, CHUNK)], actv)

    iota = lax.broadcasted_iota(jnp.int32, (L,), 0)

    def _perm(x, idx):
        return x.at[idx].get(mode="promise_in_bounds")

    def _xmax(x):
        for sh in (8, 4, 2, 1):
            x = jnp.maximum(x, _perm(x, iota ^ sh))
        return x

    def _xmin(x):
        for sh in (8, 4, 2, 1):
            x = jnp.minimum(x, _perm(x, iota ^ sh))
        return x

    def init_xyxy(k, c):
        sl = pl.ds(k * L, L)
        cx = x1v[sl] * 640.0
        cy = y1v[sl] * 640.0
        w = x2v[sl] * 100.0 + 2.0
        h = y2v[sl] * 100.0 + 2.0
        x1v[sl] = cx - w * 0.5
        y1v[sl] = cy - h * 0.5
        x2v[sl] = cx + w * 0.5
        y2v[sl] = cy + h * 0.5
        return c
    lax.fori_loop(0, PAD_N // L, init_xyxy, 0)

    @plsc.parallel_loop(0, STEPS, unroll=4)
    def _init_chunk(k):
        sl = pl.ds(k * L, L)
        s = actv[sl]
        actv[sl] = jnp.where(s > CONF_THRES, s, -1.0)

    bv0 = jnp.full((L,), -3e38, jnp.float32)
    bk0 = jnp.zeros((L,), jnp.int32)

    def fused_sweep(v, j, bx1, by1, bx2, by2, a1):
        @plsc.parallel_loop(0, STEPS, step=4,
                            carry=((bv0, bk0),) * 4, unroll=1)
        def chains(k0, am):
            out = []
            for c in range(4):
                bv2, bk2 = am[c]
                k = k0 + c
                sl = pl.ds(k * L, L)
                gsl = pl.ds(base + k * L, L)
                x1 = x1v[gsl]
                y1 = y1v[gsl]
                x2 = x2v[gsl]
                y2 = y2v[gsl]
                xx1 = jnp.maximum(bx1, x1)
                yy1 = jnp.maximum(by1, y1)
                xx2 = jnp.minimum(bx2, x2)
                yy2 = jnp.minimum(by2, y2)
                inter = (jnp.maximum(xx2 - xx1, 0.0)
                         * jnp.maximum(yy2 - yy1, 0.0))
                a2 = (x2 - x1) * (y2 - y1)
                iou = inter / (a1 + a2 - inter + 1e-7)
                g = base + k * L + iota
                sup = jnp.logical_and(
                    jnp.logical_or(iou > IOU_THRES, g == j), v)
                nact = jnp.where(sup, -1.0, actv[sl])
                actv[sl] = nact
                upd = nact > bv2
                out.append((jnp.where(upd, nact, bv2),
                            jnp.where(upd, k, bk2)))
            return tuple(out)

        def comb(p, q):
            bvp, bkp = p
            bvq, bkq = q
            upd = (bvq > bvp) | ((bvq == bvp) & (bkq < bkp))
            return (jnp.where(upd, bvq, bvp), jnp.where(upd, bkq, bkp))
        (p0, p1, p2, p3) = chains
        return comb(comb(p0, p1), comb(p2, p3))

    vfalse = iota < 0
    j0 = jnp.zeros((L,), jnp.int32)
    c0 = plsc.load_gather(x1v, [j0])
    am_init = fused_sweep(vfalse, j0, c0, c0, c0, c0, c0)

    def round_body(i, am):
        bv, bk = am
        gidx_lane = base + bk * L + iota
        m_loc = _xmax(bv)
        j_loc = _xmin(jnp.where(bv == m_loc, gidx_lane, jnp.int32(2**30)))

        pubv[:] = jnp.where(iota == 0, m_loc,
                  jnp.where(iota == 1, j_loc.astype(jnp.float32), 0.0))
        par = (i & 1) * (NSUB * L)
        pltpu.sync_copy(pubv, shared.at[pl.ds(par + sid * L, L)])
        plsc.subcore_barrier()
        pltpu.sync_copy(shared.at[pl.ds(par, NSUB * L)], mrgv)
        vals = plsc.load_gather(mrgv, [iota * L])
        idxs = plsc.load_gather(mrgv, [iota * L + 1])
        best_m = _xmax(vals)
        j = _xmin(jnp.where(vals == best_m, idxs, 3e38)).astype(jnp.int32)
        v = best_m > 0.0

        bx1 = plsc.load_gather(x1v, [j])
        by1 = plsc.load_gather(y1v, [j])
        bx2 = plsc.load_gather(x2v, [j])
        by2 = plsc.load_gather(y2v, [j])
        a1 = (bx2 - bx1) * (by2 - by1)

        @pl.when(first)
        def _():
            vf = jnp.where(v, 1.0, 0.0)
            row = jnp.where(iota == 0, bx1 * vf,
                  jnp.where(iota == 1, by1 * vf,
                  jnp.where(iota == 2, bx2 * vf,
                  jnp.where(iota == 3, by2 * vf,
                  jnp.where(iota == 4, best_m * vf, 0.0)))))
            plsc.store_scatter(outv, [i * L + iota], row)

        am_next = fused_sweep(v, j, bx1, by1, bx2, by2, a1)
        return am_next

    lax.fori_loop(0, MAX_DET, round_body, am_init)

    @pl.when(first)
    def _():
        pltpu.sync_copy(outv, out_hbm)


@jax.jit
def kernel(boxes, scores):
    pad = PAD_N - N_BOXES
    bp = jnp.pad(boxes, ((0, pad), (0, 0)))
    sp = jnp.pad(scores, (0, pad))
    f32 = jnp.float32
    mesh = plsc.VectorSubcoreMesh(core_axis_name="c", subcore_axis_name="s")
    k = pl.kernel(
        _nms_sc,
        mesh=mesh,
        compiler_params=pltpu.CompilerParams(needs_layout_passes=False),
        out_type=jax.ShapeDtypeStruct((MAX_DET * L,), f32),
        scratch_types=[
            pltpu.VMEM((PAD_N,), f32),
            pltpu.VMEM((PAD_N,), f32),
            pltpu.VMEM((PAD_N,), f32),
            pltpu.VMEM((PAD_N,), f32),
            pltpu.VMEM((CHUNK,), f32),
            pltpu.VMEM((L,), f32),
            pltpu.VMEM((NSUB * L,), f32),
            pltpu.VMEM((MAX_DET * L,), f32),
            pltpu.VMEM_SHARED((2 * NSUB * L,), f32),
        ],
    )
    out = k(bp[:, 0], bp[:, 1], bp[:, 2], bp[:, 3], sp)
    return out.reshape(MAX_DET, L)[:, :5]


# speedup vs baseline: 1.0327x; 1.0133x over previous
import jax
import jax.numpy as jnp
from jax import lax
from jax.experimental import pallas as pl
from jax.experimental.pallas import tpu as pltpu
from jax.experimental.pallas import tpu_sc as plsc

CONF_THRES = 0.25
IOU_THRES = 0.45
MAX_DET = 300
N_BOXES = 20000
PAD_N = 20480
NSUB = 16
CHUNK = PAD_N // NSUB
L = 16
STEPS = CHUNK // L


def _nms_sc(cx_hbm, cy_hbm, w_hbm, h_hbm, s_hbm, out_hbm,
            x1v, y1v, x2v, y2v, actv, pubv, mrgv, outv, shared):
    cid = lax.axis_index("c")
    sid = lax.axis_index("s")
    base = sid * CHUNK
    first = jnp.logical_and(cid == 0, sid == 0)

    pltpu.sync_copy(cx_hbm, x1v)
    pltpu.sync_copy(cy_hbm, y1v)
    pltpu.sync_copy(w_hbm, x2v)
    pltpu.sync_copy(h_hbm, y2v)
    pltpu.sync_copy(s_hbm.at[pl.ds(base, CHUNK)], actv)

    iota = lax.broadcasted_iota(jnp.int32, (L,), 0)

    def _perm(x, idx):
        return x.at[idx].get(mode="promise_in_bounds")

    def _xmax(x):
        for sh in (8, 4, 2, 1):
            x = jnp.maximum(x, _perm(x, iota ^ sh))
        return x

    def _xmin(x):
        for sh in (8, 4, 2, 1):
            x = jnp.minimum(x, _perm(x, iota ^ sh))
        return x

    def init_xyxy(k, c):
        sl = pl.ds(k * L, L)
        cx = x1v[sl] * 640.0
        cy = y1v[sl] * 640.0
        w = x2v[sl] * 100.0 + 2.0
        h = y2v[sl] * 100.0 + 2.0
        x1v[sl] = cx - w * 0.5
        y1v[sl] = cy - h * 0.5
        x2v[sl] = cx + w * 0.5
        y2v[sl] = cy + h * 0.5
        return c
    lax.fori_loop(0, PAD_N // L, init_xyxy, 0)

    @plsc.parallel_loop(0, STEPS, unroll=4)
    def _init_chunk(k):
        sl = pl.ds(k * L, L)
        s = actv[sl]
        actv[sl] = jnp.where(s > CONF_THRES, s, -1.0)

    bv0 = jnp.full((L,), -3e38, jnp.float32)
    bk0 = jnp.zeros((L,), jnp.int32)

    def fused_sweep(v, j, bx1, by1, bx2, by2, a1):
        @plsc.parallel_loop(0, STEPS, step=4,
                            carry=((bv0, bk0),) * 4, unroll=1)
        def chains(k0, am):
            out = []
            for c in range(4):
                bv2, bk2 = am[c]
                k = k0 + c
                sl = pl.ds(k * L, L)
                gsl = pl.ds(base + k * L, L)
                x1 = x1v[gsl]
                y1 = y1v[gsl]
                x2 = x2v[gsl]
                y2 = y2v[gsl]
                xx1 = jnp.maximum(bx1, x1)
                yy1 = jnp.maximum(by1, y1)
                xx2 = jnp.minimum(bx2, x2)
                yy2 = jnp.minimum(by2, y2)
                inter = (jnp.maximum(xx2 - xx1, 0.0)
                         * jnp.maximum(yy2 - yy1, 0.0))
                a2 = (x2 - x1) * (y2 - y1)
                iou = inter / (a1 + a2 - inter + 1e-7)
                g = base + k * L + iota
                sup = jnp.logical_and(
                    jnp.logical_or(iou > IOU_THRES, g == j), v)
                nact = jnp.where(sup, -1.0, actv[sl])
                actv[sl] = nact
                upd = nact > bv2
                out.append((jnp.where(upd, nact, bv2),
                            jnp.where(upd, k, bk2)))
            return tuple(out)

        def comb(p, q):
            bvp, bkp = p
            bvq, bkq = q
            upd = (bvq > bvp) | ((bvq == bvp) & (bkq < bkp))
            return (jnp.where(upd, bvq, bvp), jnp.where(upd, bkq, bkp))
        (p0, p1, p2, p3) = chains
        return comb(comb(p0, p1), comb(p2, p3))

    vfalse = iota < 0
    j0 = jnp.zeros((L,), jnp.int32)
    c0 = plsc.load_gather(x1v, [j0])
    am_init = fused_sweep(vfalse, j0, c0, c0, c0, c0, c0)

    def round_body(i, am):
        bv, bk = am
        gidx_lane = base + bk * L + iota
        m_loc = _xmax(bv)
        j_loc = _xmin(jnp.where(bv == m_loc, gidx_lane, jnp.int32(2**30)))

        pubv[:] = jnp.where(iota == 0, m_loc,
                  jnp.where(iota == 1, j_loc.astype(jnp.float32), 0.0))
        par = (i & 1) * (NSUB * L)
        pltpu.sync_copy(pubv, shared.at[pl.ds(par + sid * L, L)])
        plsc.subcore_barrier()
        pltpu.sync_copy(shared.at[pl.ds(par, NSUB * L)], mrgv)
        vals = plsc.load_gather(mrgv, [iota * L])
        idxs = plsc.load_gather(mrgv, [iota * L + 1])
        best_m = _xmax(vals)
        j = _xmin(jnp.where(vals == best_m, idxs, 3e38)).astype(jnp.int32)
        v = best_m > 0.0

        bx1 = plsc.load_gather(x1v, [j])
        by1 = plsc.load_gather(y1v, [j])
        bx2 = plsc.load_gather(x2v, [j])
        by2 = plsc.load_gather(y2v, [j])
        a1 = (bx2 - bx1) * (by2 - by1)

        am_next = fused_sweep(v, j, bx1, by1, bx2, by2, a1)

        @pl.when(first)
        def _():
            vf = jnp.where(v, 1.0, 0.0)
            row = jnp.where(iota == 0, bx1 * vf,
                  jnp.where(iota == 1, by1 * vf,
                  jnp.where(iota == 2, bx2 * vf,
                  jnp.where(iota == 3, by2 * vf,
                  jnp.where(iota == 4, best_m * vf, 0.0)))))
            plsc.store_scatter(outv, [i * L + iota], row)
        return am_next

    lax.fori_loop(0, MAX_DET, round_body, am_init)

    @pl.when(first)
    def _():
        pltpu.sync_copy(outv, out_hbm)


@jax.jit
def kernel(boxes, scores):
    pad = PAD_N - N_BOXES
    bp = jnp.pad(boxes, ((0, pad), (0, 0)))
    sp = jnp.pad(scores, (0, pad))
    f32 = jnp.float32
    mesh = plsc.VectorSubcoreMesh(core_axis_name="c", subcore_axis_name="s")
    k = pl.kernel(
        _nms_sc,
        mesh=mesh,
        compiler_params=pltpu.CompilerParams(needs_layout_passes=False),
        out_type=jax.ShapeDtypeStruct((MAX_DET * L,), f32),
        scratch_types=[
            pltpu.VMEM((PAD_N,), f32),
            pltpu.VMEM((PAD_N,), f32),
            pltpu.VMEM((PAD_N,), f32),
            pltpu.VMEM((PAD_N,), f32),
            pltpu.VMEM((CHUNK,), f32),
            pltpu.VMEM((L,), f32),
            pltpu.VMEM((NSUB * L,), f32),
            pltpu.VMEM((MAX_DET * L,), f32),
            pltpu.VMEM_SHARED((2 * NSUB * L,), f32),
        ],
    )
    out = k(bp[:, 0], bp[:, 1], bp[:, 2], bp[:, 3], sp)
    return out.reshape(MAX_DET, L)[:, :5]
